# baseline (speedup 1.0000x reference)
import functools

import jax
import jax.numpy as jnp
from jax import lax
from jax.experimental import pallas as pl
from jax.experimental.pallas import tpu as pltpu
from jax.experimental.pallas import tpu_sc as plsc

BATCH = 16384
EMB = 128
LANES = 16
NUM_CORES = 2
NUM_SUBCORES = 16
NUM_WORKERS = NUM_CORES * NUM_SUBCORES
BPW = BATCH // NUM_WORKERS
CHUNK = 128
NCHUNK = BPW // CHUNK
DCHUNKS = EMB // LANES
GROUPS = CHUNK // LANES


def _compute_chunk(rows1, rowsr, rows2, accs, out_v, ck):
  def group(g, carry):
    for i in range(LANES):
      row = g * LANES + i
      acc = (rows1[row, pl.ds(0, LANES)]
             * rowsr[row, pl.ds(0, LANES)]
             * rows2[row, pl.ds(0, LANES)])
      for j in range(1, DCHUNKS):
        acc = acc + (rows1[row, pl.ds(j * LANES, LANES)]
                     * rowsr[row, pl.ds(j * LANES, LANES)]
                     * rows2[row, pl.ds(j * LANES, LANES)])
      accs[pl.ds(i * LANES, LANES)] = acc
    iota = lax.iota(jnp.int32, LANES)
    rowbase = iota * LANES
    ssum = plsc.load_gather(accs, [rowbase + iota])
    for d in range(1, LANES):
      col = jnp.bitwise_and(iota + d, LANES - 1)
      ssum = ssum + plsc.load_gather(accs, [rowbase + col])
    out_v[pl.ds(ck * CHUNK + g * LANES, LANES)] = ssum
    return carry

  lax.fori_loop(0, GROUPS, group, 0)


def _score_body(x_hbm, ent_hbm, rel_hbm, out_hbm,
                xflat, idx1, idxr, idx2,
                rows1a, rowsra, rows2a, rows1b, rowsrb, rows2b,
                accs, out_v, sem0, sem1):
  wid = lax.axis_index("s") * NUM_CORES + lax.axis_index("c")
  base = wid * BPW

  pltpu.sync_copy(x_hbm.at[pl.ds(base * 3, BPW * 3)], xflat)
  iota = lax.iota(jnp.int32, LANES)
  iota3 = iota * 3
  for g in range(BPW // LANES):
    ck, o = divmod(g, GROUPS)
    flat = iota3 + g * (3 * LANES)
    idx1[ck, pl.ds(o * LANES, LANES)] = plsc.load_gather(xflat, [flat])
    idxr[ck, pl.ds(o * LANES, LANES)] = plsc.load_gather(xflat, [flat + 1])
    idx2[ck, pl.ds(o * LANES, LANES)] = plsc.load_gather(xflat, [flat + 2])

  rows1 = (rows1a, rows1b)
  rowsr = (rowsra, rowsrb)
  rows2 = (rows2a, rows2b)
  sems = (sem0, sem1)

  def fire(ck):
    buf = ck % 2
    return (
        pltpu.async_copy(ent_hbm.at[idx1.at[ck]], rows1[buf], sems[buf]),
        pltpu.async_copy(rel_hbm.at[idxr.at[ck]], rowsr[buf], sems[buf]),
        pltpu.async_copy(ent_hbm.at[idx2.at[ck]], rows2[buf], sems[buf]),
    )

  pending = fire(0)
  for ck in range(NCHUNK):
    buf = ck % 2
    cur = pending
    if ck + 1 < NCHUNK:
      pending = fire(ck + 1)
    for h in cur:
      h.wait()
    _compute_chunk(rows1[buf], rowsr[buf], rows2[buf], accs, out_v, ck)

  pltpu.sync_copy(out_v, out_hbm.at[pl.ds(base, BPW)])


@functools.partial(
    pl.kernel,
    out_type=jax.ShapeDtypeStruct((BATCH,), jnp.float32),
    mesh=plsc.VectorSubcoreMesh(core_axis_name="c", subcore_axis_name="s"),
    scratch_types=[
        pltpu.VMEM((BPW * 3,), jnp.int32),
        pltpu.VMEM((NCHUNK, CHUNK), jnp.int32),
        pltpu.VMEM((NCHUNK, CHUNK), jnp.int32),
        pltpu.VMEM((NCHUNK, CHUNK), jnp.int32),
        pltpu.VMEM((CHUNK, EMB), jnp.float32),
        pltpu.VMEM((CHUNK, EMB), jnp.float32),
        pltpu.VMEM((CHUNK, EMB), jnp.float32),
        pltpu.VMEM((CHUNK, EMB), jnp.float32),
        pltpu.VMEM((CHUNK, EMB), jnp.float32),
        pltpu.VMEM((CHUNK, EMB), jnp.float32),
        pltpu.VMEM((LANES * LANES,), jnp.float32),
        pltpu.VMEM((BPW,), jnp.float32),
        pltpu.SemaphoreType.DMA,
        pltpu.SemaphoreType.DMA,
    ],
    compiler_params=pltpu.CompilerParams(needs_layout_passes=False),
)
def _score_kernel(x_hbm, ent, rel, out,
                  xflat, idx1, idxr, idx2,
                  rows1a, rowsra, rows2a, rows1b, rowsrb, rows2b,
                  accs, out_v, sem0, sem1):
  _score_body(x_hbm, ent, rel, out,
              xflat, idx1, idxr, idx2,
              rows1a, rowsra, rows2a, rows1b, rowsrb, rows2b,
              accs, out_v, sem0, sem1)


@jax.jit
def kernel(x, entity_emb, relation_emb):
  return _score_kernel(x.reshape(BATCH * 3), entity_emb, relation_emb)


# speedup vs baseline: 1.0522x; 1.0522x over previous
import functools

import jax
import jax.numpy as jnp
from jax import lax
from jax.experimental import pallas as pl
from jax.experimental.pallas import tpu as pltpu
from jax.experimental.pallas import tpu_sc as plsc

BATCH = 16384
EMB = 128
LANES = 16
NUM_CORES = 2
NUM_SUBCORES = 16
NUM_WORKERS = NUM_CORES * NUM_SUBCORES
BPW = BATCH // NUM_WORKERS
CHUNK = 128
NCHUNK = BPW // CHUNK
DCHUNKS = EMB // LANES
GROUPS = CHUNK // LANES


def _compute_chunk(rows1, rowsr, rows2, accs, out_v, ck):
  def row_body(i, carry):
    acc = (rows1[i, pl.ds(0, LANES)]
           * rowsr[i, pl.ds(0, LANES)]
           * rows2[i, pl.ds(0, LANES)])
    for j in range(1, DCHUNKS):
      acc = acc + (rows1[i, pl.ds(j * LANES, LANES)]
                   * rowsr[i, pl.ds(j * LANES, LANES)]
                   * rows2[i, pl.ds(j * LANES, LANES)])
    accs[pl.ds(i * LANES, LANES)] = acc
    return carry

  lax.fori_loop(0, CHUNK, row_body, 0)

  def group(g, carry):
    iota = lax.iota(jnp.int32, LANES)
    rowbase = (g * LANES + iota) * LANES
    ssum = plsc.load_gather(accs, [rowbase + iota])
    for d in range(1, LANES):
      col = jnp.bitwise_and(iota + d, LANES - 1)
      ssum = ssum + plsc.load_gather(accs, [rowbase + col])
    out_v[pl.ds(ck * CHUNK + g * LANES, LANES)] = ssum
    return carry

  lax.fori_loop(0, GROUPS, group, 0)


def _score_body(x_hbm, ent_hbm, rel_hbm, out_hbm,
                xflat, idx1, idxr, idx2,
                rows1a, rowsra, rows2a, rows1b, rowsrb, rows2b,
                accs, out_v, sem0, sem1):
  wid = lax.axis_index("s") * NUM_CORES + lax.axis_index("c")
  base = wid * BPW

  pltpu.sync_copy(x_hbm.at[pl.ds(base * 3, BPW * 3)], xflat)
  iota = lax.iota(jnp.int32, LANES)
  iota3 = iota * 3

  def split_body(g, carry):
    ck = g // GROUPS
    o = g - ck * GROUPS
    flat = iota3 + g * (3 * LANES)
    idx1[ck, pl.ds(o * LANES, LANES)] = plsc.load_gather(xflat, [flat])
    idxr[ck, pl.ds(o * LANES, LANES)] = plsc.load_gather(xflat, [flat + 1])
    idx2[ck, pl.ds(o * LANES, LANES)] = plsc.load_gather(xflat, [flat + 2])
    return carry

  lax.fori_loop(0, BPW // LANES, split_body, 0)

  rows1 = (rows1a, rows1b)
  rowsr = (rowsra, rowsrb)
  rows2 = (rows2a, rows2b)
  sems = (sem0, sem1)

  def fire(ck):
    buf = ck % 2
    return (
        pltpu.async_copy(ent_hbm.at[idx1.at[ck]], rows1[buf], sems[buf]),
        pltpu.async_copy(rel_hbm.at[idxr.at[ck]], rowsr[buf], sems[buf]),
        pltpu.async_copy(ent_hbm.at[idx2.at[ck]], rows2[buf], sems[buf]),
    )

  pending = fire(0)
  for ck in range(NCHUNK):
    buf = ck % 2
    cur = pending
    if ck + 1 < NCHUNK:
      pending = fire(ck + 1)
    for h in cur:
      h.wait()
    _compute_chunk(rows1[buf], rowsr[buf], rows2[buf], accs, out_v, ck)

  pltpu.sync_copy(out_v, out_hbm.at[pl.ds(base, BPW)])


@functools.partial(
    pl.kernel,
    out_type=jax.ShapeDtypeStruct((BATCH,), jnp.float32),
    mesh=plsc.VectorSubcoreMesh(core_axis_name="c", subcore_axis_name="s"),
    scratch_types=[
        pltpu.VMEM((BPW * 3,), jnp.int32),
        pltpu.VMEM((NCHUNK, CHUNK), jnp.int32),
        pltpu.VMEM((NCHUNK, CHUNK), jnp.int32),
        pltpu.VMEM((NCHUNK, CHUNK), jnp.int32),
        pltpu.VMEM((CHUNK, EMB), jnp.float32),
        pltpu.VMEM((CHUNK, EMB), jnp.float32),
        pltpu.VMEM((CHUNK, EMB), jnp.float32),
        pltpu.VMEM((CHUNK, EMB), jnp.float32),
        pltpu.VMEM((CHUNK, EMB), jnp.float32),
        pltpu.VMEM((CHUNK, EMB), jnp.float32),
        pltpu.VMEM((CHUNK * LANES,), jnp.float32),
        pltpu.VMEM((BPW,), jnp.float32),
        pltpu.SemaphoreType.DMA,
        pltpu.SemaphoreType.DMA,
    ],
    compiler_params=pltpu.CompilerParams(needs_layout_passes=False),
)
def _score_kernel(x_hbm, ent, rel, out,
                  xflat, idx1, idxr, idx2,
                  rows1a, rowsra, rows2a, rows1b, rowsrb, rows2b,
                  accs, out_v, sem0, sem1):
  _score_body(x_hbm, ent, rel, out,
              xflat, idx1, idxr, idx2,
              rows1a, rowsra, rows2a, rows1b, rowsrb, rows2b,
              accs, out_v, sem0, sem1)


@jax.jit
def kernel(x, entity_emb, relation_emb):
  return _score_kernel(x.reshape(BATCH * 3), entity_emb, relation_emb)


# speedup vs baseline: 1.3026x; 1.2379x over previous
import functools

import jax
import jax.numpy as jnp
from jax import lax
from jax.experimental import pallas as pl
from jax.experimental.pallas import tpu as pltpu
from jax.experimental.pallas import tpu_sc as plsc

BATCH = 16384
EMB = 128
LANES = 16
NUM_CORES = 2
NUM_SUBCORES = 16
NUM_WORKERS = NUM_CORES * NUM_SUBCORES
BPW = BATCH // NUM_WORKERS
CHUNK = 128
NCHUNK = BPW // CHUNK
DCHUNKS = EMB // LANES
GROUPS = CHUNK // LANES


def _compute_chunk(rows1, rowsr, rows2, accs, out_v, ck):
  def row_body(i, carry):
    acc = (rows1[i, pl.ds(0, LANES)]
           * rowsr[i, pl.ds(0, LANES)]
           * rows2[i, pl.ds(0, LANES)])
    for j in range(1, DCHUNKS):
      acc = acc + (rows1[i, pl.ds(j * LANES, LANES)]
                   * rowsr[i, pl.ds(j * LANES, LANES)]
                   * rows2[i, pl.ds(j * LANES, LANES)])
    accs[pl.ds(i * LANES, LANES)] = acc
    return carry

  lax.fori_loop(0, CHUNK, row_body, 0)

  def group(g, carry):
    iota = lax.iota(jnp.int32, LANES)
    rowbase = (g * LANES + iota) * LANES
    ssum = plsc.load_gather(accs, [rowbase + iota])
    for d in range(1, LANES):
      col = jnp.bitwise_and(iota + d, LANES - 1)
      ssum = ssum + plsc.load_gather(accs, [rowbase + col])
    out_v[pl.ds(ck * CHUNK + g * LANES, LANES)] = ssum
    return carry

  lax.fori_loop(0, GROUPS, group, 0)


def _score_body(e1i_hbm, ri_hbm, e2i_hbm, ent_hbm, rel_hbm, out_hbm,
                idx1, idxr, idx2,
                rows1a, rowsra, rows2a, rows1b, rowsrb, rows2b,
                accs, out_v, sem0, sem1):
  wid = lax.axis_index("s") * NUM_CORES + lax.axis_index("c")
  base = wid * BPW

  pltpu.sync_copy(e1i_hbm.at[pl.ds(wid * NCHUNK, NCHUNK)], idx1)
  pltpu.sync_copy(ri_hbm.at[pl.ds(wid * NCHUNK, NCHUNK)], idxr)
  pltpu.sync_copy(e2i_hbm.at[pl.ds(wid * NCHUNK, NCHUNK)], idx2)

  rows1 = (rows1a, rows1b)
  rowsr = (rowsra, rowsrb)
  rows2 = (rows2a, rows2b)
  sems = (sem0, sem1)

  def fire(ck):
    buf = ck % 2
    return (
        pltpu.async_copy(ent_hbm.at[idx1.at[ck]], rows1[buf], sems[buf]),
        pltpu.async_copy(rel_hbm.at[idxr.at[ck]], rowsr[buf], sems[buf]),
        pltpu.async_copy(ent_hbm.at[idx2.at[ck]], rows2[buf], sems[buf]),
    )

  pending = fire(0)
  for ck in range(NCHUNK):
    buf = ck % 2
    cur = pending
    if ck + 1 < NCHUNK:
      pending = fire(ck + 1)
    for h in cur:
      h.wait()
    _compute_chunk(rows1[buf], rowsr[buf], rows2[buf], accs, out_v, ck)

  pltpu.sync_copy(out_v, out_hbm.at[pl.ds(base, BPW)])


@functools.partial(
    pl.kernel,
    out_type=jax.ShapeDtypeStruct((BATCH,), jnp.float32),
    mesh=plsc.VectorSubcoreMesh(core_axis_name="c", subcore_axis_name="s"),
    scratch_types=[
        pltpu.VMEM((NCHUNK, CHUNK), jnp.int32),
        pltpu.VMEM((NCHUNK, CHUNK), jnp.int32),
        pltpu.VMEM((NCHUNK, CHUNK), jnp.int32),
        pltpu.VMEM((CHUNK, EMB), jnp.float32),
        pltpu.VMEM((CHUNK, EMB), jnp.float32),
        pltpu.VMEM((CHUNK, EMB), jnp.float32),
        pltpu.VMEM((CHUNK, EMB), jnp.float32),
        pltpu.VMEM((CHUNK, EMB), jnp.float32),
        pltpu.VMEM((CHUNK, EMB), jnp.float32),
        pltpu.VMEM((CHUNK * LANES,), jnp.float32),
        pltpu.VMEM((BPW,), jnp.float32),
        pltpu.SemaphoreType.DMA,
        pltpu.SemaphoreType.DMA,
    ],
    compiler_params=pltpu.CompilerParams(needs_layout_passes=False),
)
def _score_kernel(e1i, ri, e2i, ent, rel, out,
                  idx1, idxr, idx2,
                  rows1a, rowsra, rows2a, rows1b, rowsrb, rows2b,
                  accs, out_v, sem0, sem1):
  _score_body(e1i, ri, e2i, ent, rel, out,
              idx1, idxr, idx2,
              rows1a, rowsra, rows2a, rows1b, rowsrb, rows2b,
              accs, out_v, sem0, sem1)


@jax.jit
def kernel(x, entity_emb, relation_emb):
  e1i = x[:, 0].reshape(NUM_WORKERS * NCHUNK, CHUNK)
  ri = x[:, 1].reshape(NUM_WORKERS * NCHUNK, CHUNK)
  e2i = x[:, 2].reshape(NUM_WORKERS * NCHUNK, CHUNK)
  return _score_kernel(e1i, ri, e2i, entity_emb, relation_emb)
